# baseline (speedup 1.0000x reference)
import functools

import jax
import jax.numpy as jnp
from jax import lax
from jax.experimental import pallas as pl
from jax.experimental.pallas import tpu as pltpu
from jax.experimental.pallas import tpu_sc as plsc

N_TOK = 16384
K_CODES = 8192
D_DIM = 256

N_TILE = 512
K_TILE = 1024
WIN1 = 2736
WIN2 = 5472


def _bf16_round(v):
    return v.astype(jnp.bfloat16).astype(jnp.float32)


def _dist_body(x_ref, w_ref, xs_ref, ws_ref, idx_ref, loss_ref,
               m0_ref, a0_ref, m1_ref, a1_ref, m2_ref, a2_ref):
    j = pl.program_id(1)
    nk = pl.num_programs(1)
    i = pl.program_id(0)

    x = x_ref[...]
    w = w_ref[...]
    xw = lax.dot_general(
        x.astype(jnp.bfloat16), w.astype(jnp.bfloat16),
        (((1,), (1,)), ((), ())),
        preferred_element_type=jnp.float32,
    )
    x_sq = xs_ref[...]
    w_sq = ws_ref[...]
    d2 = (x_sq + w_sq) - 2.0 * xw
    d = jnp.sqrt(jnp.maximum(d2, 0.0))

    col = lax.broadcasted_iota(jnp.int32, d.shape, 1) + j * K_TILE

    def tile_minarg(vals):
        lmin = jnp.min(vals, axis=1, keepdims=True)
        larg = jnp.min(jnp.where(vals == lmin, col, K_CODES),
                       axis=1, keepdims=True)
        return lmin, larg

    def set_pair(mref, aref, vals):
        lmin, larg = tile_minarg(vals)
        mref[...] = lmin
        aref[...] = larg

    def merge_pair(mref, aref, vals):
        lmin, larg = tile_minarg(vals)
        better = lmin < mref[...]
        mref[...] = jnp.where(better, lmin, mref[...])
        aref[...] = jnp.where(better, larg, aref[...])

    @pl.when(j == 0)
    def _():
        set_pair(m0_ref, a0_ref, d)

    @pl.when(j == 1)
    def _():
        merge_pair(m0_ref, a0_ref, d)

    @pl.when(j == 2)
    def _():
        in0 = col < WIN1
        merge_pair(m0_ref, a0_ref, jnp.where(in0, d, jnp.inf))
        set_pair(m1_ref, a1_ref, jnp.where(in0, jnp.inf, d))

    @pl.when((j == 3) | (j == 4))
    def _():
        merge_pair(m1_ref, a1_ref, d)

    @pl.when(j == 5)
    def _():
        in1 = col < WIN2
        merge_pair(m1_ref, a1_ref, jnp.where(in1, d, jnp.inf))
        set_pair(m2_ref, a2_ref, jnp.where(in1, jnp.inf, d))

    @pl.when(j == 6)
    def _():
        merge_pair(m2_ref, a2_ref, d)

    @pl.when(j == nk - 1)
    def _():
        merge_pair(m2_ref, a2_ref, d)
        run_b = _bf16_round(m0_ref[...])
        idx = a0_ref[...]
        lossv = m0_ref[...]
        win1 = m1_ref[...] < run_b
        idx = jnp.where(win1, a1_ref[...], idx)
        lossv = jnp.where(win1, m1_ref[...], lossv)
        run_b = jnp.where(win1, _bf16_round(m1_ref[...]), run_b)
        win2 = m2_ref[...] < run_b
        idx = jnp.where(win2, a2_ref[...], idx)
        lossv = jnp.where(win2, m2_ref[...], lossv)
        idx_ref[...] = idx
        part = jnp.sum(lossv * lossv, keepdims=True)

        @pl.when(i == 0)
        def _():
            loss_ref[...] = part

        @pl.when(i > 0)
        def _():
            loss_ref[...] = loss_ref[...] + part


def _dist_argmin(flat_x, W):
    ni = N_TOK // N_TILE
    nk = K_CODES // K_TILE
    assert nk == 8 and K_TILE == 1024
    xs = jnp.sum(flat_x * flat_x, axis=1, keepdims=True)
    ws = jnp.sum(W * W, axis=1)[None, :]
    return pl.pallas_call(
        _dist_body,
        grid=(ni, nk),
        in_specs=[
            pl.BlockSpec((N_TILE, D_DIM), lambda i, j: (i, 0)),
            pl.BlockSpec((K_TILE, D_DIM), lambda i, j: (j, 0)),
            pl.BlockSpec((N_TILE, 1), lambda i, j: (i, 0)),
            pl.BlockSpec((1, K_TILE), lambda i, j: (0, j)),
        ],
        out_specs=[
            pl.BlockSpec((N_TILE, 1), lambda i, j: (i, 0)),
            pl.BlockSpec((1, 1), lambda i, j: (0, 0)),
        ],
        out_shape=[
            jax.ShapeDtypeStruct((N_TOK, 1), jnp.int32),
            jax.ShapeDtypeStruct((1, 1), jnp.float32),
        ],
        scratch_shapes=[
            pltpu.VMEM((N_TILE, 1), jnp.float32),
            pltpu.VMEM((N_TILE, 1), jnp.int32),
            pltpu.VMEM((N_TILE, 1), jnp.float32),
            pltpu.VMEM((N_TILE, 1), jnp.int32),
            pltpu.VMEM((N_TILE, 1), jnp.float32),
            pltpu.VMEM((N_TILE, 1), jnp.int32),
        ],
    )(flat_x, W, xs, ws)


_NC, _NS = 2, 16
_NW = _NC * _NS
_ROWS_PER_W = N_TOK // _NW
_CHUNK = 256


@functools.cache
def _make_gather():
    @functools.partial(
        pl.kernel,
        mesh=plsc.VectorSubcoreMesh(core_axis_name="c", subcore_axis_name="s"),
        out_type=jax.ShapeDtypeStruct((N_TOK, D_DIM), jnp.float32),
        scratch_types=[
            pltpu.VMEM((_ROWS_PER_W,), jnp.int32),
            pltpu.VMEM((_CHUNK, D_DIM), jnp.float32),
            pltpu.SemaphoreType.DMA,
        ],
    )
    def _gather_rows(table_hbm, idx_hbm, out_hbm, idx_v, rows_v, sem):
        wid = lax.axis_index("s") * _NC + lax.axis_index("c")
        base = wid * _ROWS_PER_W
        pltpu.sync_copy(idx_hbm.at[pl.ds(base, _ROWS_PER_W)], idx_v)
        for c in range(_ROWS_PER_W // _CHUNK):
            pltpu.async_copy(
                table_hbm.at[idx_v.at[pl.ds(c * _CHUNK, _CHUNK)]], rows_v, sem
            ).wait()
            pltpu.sync_copy(rows_v, out_hbm.at[pl.ds(base + c * _CHUNK, _CHUNK)])

    return _gather_rows


def kernel(x, W):
    B, T, D = x.shape
    flat_x = x.reshape(B * T, D)
    idx2d, loss_sum = _dist_argmin(flat_x, W)
    q = _make_gather()(W, idx2d.reshape(-1))
    loss = (1.0 + 0.25) * loss_sum[0, 0] / (B * T * D)
    return q.reshape(B, T, D), loss


# speedup vs baseline: 1.2047x; 1.2047x over previous
import functools

import jax
import jax.numpy as jnp
from jax import lax
from jax.experimental import pallas as pl
from jax.experimental.pallas import tpu as pltpu
from jax.experimental.pallas import tpu_sc as plsc

N_TOK = 16384
K_CODES = 8192
D_DIM = 256

N_TILE = 512
K_TILE = 1024
WIN1 = 2736
WIN2 = 5472


def _bf16_round(v):
    return v.astype(jnp.bfloat16).astype(jnp.float32)


def _dist_body(x_ref, w_ref, xs_ref, ws_ref, idx_ref, loss_ref):
    i = pl.program_id(0)
    nk = K_CODES // K_TILE

    x16 = x_ref[...].astype(jnp.bfloat16)
    x_sq = xs_ref[...]

    def tile_minarg(vals, col):
        lmin = jnp.min(vals, axis=1, keepdims=True)
        larg = jnp.min(jnp.where(vals == lmin, col, K_CODES),
                       axis=1, keepdims=True)
        return lmin, larg

    wins = [None, None, None]

    def merge(slot, lmin, larg):
        if wins[slot] is None:
            wins[slot] = (lmin, larg)
        else:
            m, a = wins[slot]
            better = lmin < m
            wins[slot] = (jnp.where(better, lmin, m),
                          jnp.where(better, larg, a))

    for j in range(nk):
        w = w_ref[pl.ds(j * K_TILE, K_TILE), :]
        xw = lax.dot_general(
            x16, w.astype(jnp.bfloat16),
            (((1,), (1,)), ((), ())),
            preferred_element_type=jnp.float32,
        )
        w_sq = ws_ref[:, pl.ds(j * K_TILE, K_TILE)]
        d2 = (x_sq + w_sq) - 2.0 * xw
        d = jnp.sqrt(jnp.maximum(d2, 0.0))
        col = lax.broadcasted_iota(jnp.int32, d.shape, 1) + j * K_TILE

        lo, hi = j * K_TILE, (j + 1) * K_TILE
        if hi <= WIN1:
            merge(0, *tile_minarg(d, col))
        elif lo < WIN1:
            in0 = col < WIN1
            merge(0, *tile_minarg(jnp.where(in0, d, jnp.inf), col))
            merge(1, *tile_minarg(jnp.where(in0, jnp.inf, d), col))
        elif hi <= WIN2:
            merge(1, *tile_minarg(d, col))
        elif lo < WIN2:
            in1 = col < WIN2
            merge(1, *tile_minarg(jnp.where(in1, d, jnp.inf), col))
            merge(2, *tile_minarg(jnp.where(in1, jnp.inf, d), col))
        else:
            merge(2, *tile_minarg(d, col))

    (m0, a0), (m1, a1), (m2, a2) = wins
    run_b = _bf16_round(m0)
    idx = a0
    lossv = m0
    win1 = m1 < run_b
    idx = jnp.where(win1, a1, idx)
    lossv = jnp.where(win1, m1, lossv)
    run_b = jnp.where(win1, _bf16_round(m1), run_b)
    win2 = m2 < run_b
    idx = jnp.where(win2, a2, idx)
    lossv = jnp.where(win2, m2, lossv)
    idx_ref[...] = idx
    part = jnp.sum(lossv * lossv, keepdims=True)

    @pl.when(i == 0)
    def _():
        loss_ref[...] = part

    @pl.when(i > 0)
    def _():
        loss_ref[...] = loss_ref[...] + part


def _dist_argmin(flat_x, W):
    ni = N_TOK // N_TILE
    assert K_CODES % K_TILE == 0
    xs = jnp.sum(flat_x * flat_x, axis=1, keepdims=True)
    ws = jnp.sum(W * W, axis=1)[None, :]
    return pl.pallas_call(
        _dist_body,
        grid=(ni,),
        in_specs=[
            pl.BlockSpec((N_TILE, D_DIM), lambda i: (i, 0)),
            pl.BlockSpec((K_CODES, D_DIM), lambda i: (0, 0)),
            pl.BlockSpec((N_TILE, 1), lambda i: (i, 0)),
            pl.BlockSpec((1, K_CODES), lambda i: (0, 0)),
        ],
        out_specs=[
            pl.BlockSpec((N_TILE, 1), lambda i: (i, 0)),
            pl.BlockSpec((1, 1), lambda i: (0, 0)),
        ],
        out_shape=[
            jax.ShapeDtypeStruct((N_TOK, 1), jnp.int32),
            jax.ShapeDtypeStruct((1, 1), jnp.float32),
        ],
    )(flat_x, W, xs, ws)


_NC, _NS = 2, 16
_NW = _NC * _NS
_ROWS_PER_W = N_TOK // _NW
_CHUNK = 256


@functools.cache
def _make_gather():
    @functools.partial(
        pl.kernel,
        mesh=plsc.VectorSubcoreMesh(core_axis_name="c", subcore_axis_name="s"),
        out_type=jax.ShapeDtypeStruct((N_TOK, D_DIM), jnp.float32),
        scratch_types=[
            pltpu.VMEM((_ROWS_PER_W,), jnp.int32),
            pltpu.VMEM((_CHUNK, D_DIM), jnp.float32),
            pltpu.SemaphoreType.DMA,
        ],
    )
    def _gather_rows(table_hbm, idx_hbm, out_hbm, idx_v, rows_v, sem):
        wid = lax.axis_index("s") * _NC + lax.axis_index("c")
        base = wid * _ROWS_PER_W
        pltpu.sync_copy(idx_hbm.at[pl.ds(base, _ROWS_PER_W)], idx_v)
        for c in range(_ROWS_PER_W // _CHUNK):
            pltpu.async_copy(
                table_hbm.at[idx_v.at[pl.ds(c * _CHUNK, _CHUNK)]], rows_v, sem
            ).wait()
            pltpu.sync_copy(rows_v, out_hbm.at[pl.ds(base + c * _CHUNK, _CHUNK)])

    return _gather_rows


def kernel(x, W):
    B, T, D = x.shape
    flat_x = x.reshape(B * T, D)
    idx2d, loss_sum = _dist_argmin(flat_x, W)
    q = _make_gather()(W, idx2d.reshape(-1))
    loss = (1.0 + 0.25) * loss_sum[0, 0] / (B * T * D)
    return q.reshape(B, T, D), loss


# speedup vs baseline: 1.2567x; 1.0432x over previous
import functools

import jax
import jax.numpy as jnp
from jax import lax
from jax.experimental import pallas as pl
from jax.experimental.pallas import tpu as pltpu
from jax.experimental.pallas import tpu_sc as plsc

N_TOK = 16384
K_CODES = 8192
D_DIM = 256

N_TILE = 1024
K_TILE = 1024
WIN1 = 2736
WIN2 = 5472


def _bf16_round(v):
    return v.astype(jnp.bfloat16).astype(jnp.float32)


def _dist_body(x_ref, w_ref, xs_ref, ws_ref, idx_ref, loss_ref):
    i = pl.program_id(0)
    nk = K_CODES // K_TILE

    x16 = x_ref[...].astype(jnp.bfloat16)
    x_sq = xs_ref[...]

    def tile_minarg(vals, col):
        lmin = jnp.min(vals, axis=1, keepdims=True)
        larg = jnp.min(jnp.where(vals == lmin, col, K_CODES),
                       axis=1, keepdims=True)
        return lmin, larg

    wins = [None, None, None]

    def merge(slot, lmin, larg):
        if wins[slot] is None:
            wins[slot] = (lmin, larg)
        else:
            m, a = wins[slot]
            better = lmin < m
            wins[slot] = (jnp.where(better, lmin, m),
                          jnp.where(better, larg, a))

    for j in range(nk):
        w = w_ref[pl.ds(j * K_TILE, K_TILE), :]
        xw = lax.dot_general(
            x16, w.astype(jnp.bfloat16),
            (((1,), (1,)), ((), ())),
            preferred_element_type=jnp.float32,
        )
        w_sq = ws_ref[:, pl.ds(j * K_TILE, K_TILE)]
        d2 = (x_sq + w_sq) - 2.0 * xw
        d = jnp.sqrt(jnp.maximum(d2, 0.0))
        col = lax.broadcasted_iota(jnp.int32, d.shape, 1) + j * K_TILE

        lo, hi = j * K_TILE, (j + 1) * K_TILE
        if hi <= WIN1:
            merge(0, *tile_minarg(d, col))
        elif lo < WIN1:
            in0 = col < WIN1
            merge(0, *tile_minarg(jnp.where(in0, d, jnp.inf), col))
            merge(1, *tile_minarg(jnp.where(in0, jnp.inf, d), col))
        elif hi <= WIN2:
            merge(1, *tile_minarg(d, col))
        elif lo < WIN2:
            in1 = col < WIN2
            merge(1, *tile_minarg(jnp.where(in1, d, jnp.inf), col))
            merge(2, *tile_minarg(jnp.where(in1, jnp.inf, d), col))
        else:
            merge(2, *tile_minarg(d, col))

    (m0, a0), (m1, a1), (m2, a2) = wins
    run_b = _bf16_round(m0)
    idx = a0
    lossv = m0
    win1 = m1 < run_b
    idx = jnp.where(win1, a1, idx)
    lossv = jnp.where(win1, m1, lossv)
    run_b = jnp.where(win1, _bf16_round(m1), run_b)
    win2 = m2 < run_b
    idx = jnp.where(win2, a2, idx)
    lossv = jnp.where(win2, m2, lossv)
    idx_ref[...] = idx
    part = jnp.sum(lossv * lossv, keepdims=True)

    @pl.when(i == 0)
    def _():
        loss_ref[...] = part

    @pl.when(i > 0)
    def _():
        loss_ref[...] = loss_ref[...] + part


def _dist_argmin(flat_x, W):
    ni = N_TOK // N_TILE
    assert K_CODES % K_TILE == 0
    xs = jnp.sum(flat_x * flat_x, axis=1, keepdims=True)
    ws = jnp.sum(W * W, axis=1)[None, :]
    return pl.pallas_call(
        _dist_body,
        grid=(ni,),
        in_specs=[
            pl.BlockSpec((N_TILE, D_DIM), lambda i: (i, 0)),
            pl.BlockSpec((K_CODES, D_DIM), lambda i: (0, 0)),
            pl.BlockSpec((N_TILE, 1), lambda i: (i, 0)),
            pl.BlockSpec((1, K_CODES), lambda i: (0, 0)),
        ],
        out_specs=[
            pl.BlockSpec((N_TILE, 1), lambda i: (i, 0)),
            pl.BlockSpec((1, 1), lambda i: (0, 0)),
        ],
        out_shape=[
            jax.ShapeDtypeStruct((N_TOK, 1), jnp.int32),
            jax.ShapeDtypeStruct((1, 1), jnp.float32),
        ],
    )(flat_x, W, xs, ws)


_NC, _NS = 2, 16
_NW = _NC * _NS
_ROWS_PER_W = N_TOK // _NW
_CHUNK = 256


@functools.cache
def _make_gather():
    @functools.partial(
        pl.kernel,
        mesh=plsc.VectorSubcoreMesh(core_axis_name="c", subcore_axis_name="s"),
        out_type=jax.ShapeDtypeStruct((N_TOK, D_DIM), jnp.float32),
        scratch_types=[
            pltpu.VMEM((_ROWS_PER_W,), jnp.int32),
            pltpu.VMEM((_CHUNK, D_DIM), jnp.float32),
            pltpu.SemaphoreType.DMA,
        ],
    )
    def _gather_rows(table_hbm, idx_hbm, out_hbm, idx_v, rows_v, sem):
        wid = lax.axis_index("s") * _NC + lax.axis_index("c")
        base = wid * _ROWS_PER_W
        pltpu.sync_copy(idx_hbm.at[pl.ds(base, _ROWS_PER_W)], idx_v)
        for c in range(_ROWS_PER_W // _CHUNK):
            pltpu.async_copy(
                table_hbm.at[idx_v.at[pl.ds(c * _CHUNK, _CHUNK)]], rows_v, sem
            ).wait()
            pltpu.sync_copy(rows_v, out_hbm.at[pl.ds(base + c * _CHUNK, _CHUNK)])

    return _gather_rows


def kernel(x, W):
    B, T, D = x.shape
    flat_x = x.reshape(B * T, D)
    idx2d, loss_sum = _dist_argmin(flat_x, W)
    q = _make_gather()(W, idx2d.reshape(-1))
    loss = (1.0 + 0.25) * loss_sum[0, 0] / (B * T * D)
    return q.reshape(B, T, D), loss
